# baseline (speedup 1.0000x reference)
import functools

import jax
import jax.numpy as jnp
from jax import lax
from jax.experimental import pallas as pl
from jax.experimental.pallas import tpu as pltpu
from jax.experimental.pallas import tpu_sc as plsc

_NC = 2
_NS = 16

_CHUNK = 50

_C1 = 0.999434943
_C2 = -0.491347462
_C3 = 0.287824693
_C4 = -0.134133305
_C5 = 0.0313766221


def _pack_gu(g, u):
    bg = lax.bitcast_convert_type(g, jnp.uint32)
    bu = lax.bitcast_convert_type(u, jnp.uint32)
    half = jnp.uint32(0x7FFF)
    one = jnp.uint32(1)
    rg = (bg + half + ((bg >> 16) & one)) >> 16
    ru = (bu + half + ((bu >> 16) & one)) & jnp.uint32(0xFFFF0000)
    return ru | rg


def _node_tables(x, w_in, b_in2, m1, m2):
    n, f = x.shape
    hf = f // 2
    bn = 2000
    grid = (n // bn,)

    def body(x_ref, w_ref, b_ref, m1_ref, m2_ref, init_ref, qs_ref, pr_ref):
        h = jnp.maximum(
            jnp.dot(x_ref[...], w_ref[...], preferred_element_type=jnp.float32)
            + b_ref[...], 0.0)
        init_ref[0] = h[:, :hf]
        init_ref[1] = h[:, hf:]
        t1 = jnp.dot(h, m1_ref[...], preferred_element_type=jnp.float32)
        p1 = _pack_gu(t1[:, :f], t1[:, f:])
        qs_ref[0] = p1[:, :hf]
        qs_ref[1] = p1[:, hf:]
        t2 = jnp.dot(h, m2_ref[...], preferred_element_type=jnp.float32)
        p2 = _pack_gu(t2[:, :f], t2[:, f:])
        pr_ref[0] = p2[:, :hf]
        pr_ref[1] = p2[:, hf:]

    return pl.pallas_call(
        body,
        grid=grid,
        in_specs=[
            pl.BlockSpec((bn, f), lambda i: (i, 0)),
            pl.BlockSpec((f, f), lambda i: (0, 0)),
            pl.BlockSpec((1, f), lambda i: (0, 0)),
            pl.BlockSpec((f, 2 * f), lambda i: (0, 0)),
            pl.BlockSpec((f, 2 * f), lambda i: (0, 0)),
        ],
        out_specs=[
            pl.BlockSpec((2, bn, hf), lambda i: (0, i, 0)),
            pl.BlockSpec((2, bn, hf), lambda i: (0, i, 0)),
            pl.BlockSpec((2, bn, hf), lambda i: (0, i, 0)),
        ],
        out_shape=[
            jax.ShapeDtypeStruct((2, n, hf), jnp.float32),
            jax.ShapeDtypeStruct((2, n, hf), jnp.uint32),
            jax.ShapeDtypeStruct((2, n, hf), jnp.uint32),
        ],
    )(x, w_in, b_in2, m1, m2)


def _edge_tables(attrp, w2, bc2, f):
    ep, k = attrp.shape
    n_out = w2.shape[1]
    hf = f // 2
    bp = 4000
    grid = (ep // bp,)

    def body(a_ref, w_ref, b_ref, o_ref):
        t = jnp.dot(a_ref[...].astype(jnp.bfloat16),
                    w_ref[...].astype(jnp.bfloat16),
                    preferred_element_type=jnp.float32) + b_ref[...]
        o_ref[0] = _pack_gu(t[:, 0:f], t[:, f:2 * f])
        o_ref[1] = _pack_gu(t[:, 2 * f:3 * f], t[:, 3 * f:4 * f])

    return pl.pallas_call(
        body,
        grid=grid,
        in_specs=[
            pl.BlockSpec((bp, k), lambda i: (i, 0)),
            pl.BlockSpec((k, n_out), lambda i: (0, 0)),
            pl.BlockSpec((1, n_out), lambda i: (0, 0)),
        ],
        out_specs=pl.BlockSpec((2, bp, f), lambda i: (0, i, 0)),
        out_shape=jax.ShapeDtypeStruct((2, ep, f), jnp.uint32),
    )(attrp, w2, bc2)


def _final(sc_out, w_ffw, b_ffw2):
    _, n, hf = sc_out.shape
    f = 2 * hf
    bn = 2000
    grid = (n // bn,)

    def body(a_ref, w_ref, b_ref, o_ref):
        hh = jnp.concatenate([a_ref[0], a_ref[1]], axis=-1)
        o_ref[...] = jnp.maximum(
            jnp.dot(hh, w_ref[...], preferred_element_type=jnp.float32)
            + b_ref[...], 0.0)

    return pl.pallas_call(
        body,
        grid=grid,
        in_specs=[
            pl.BlockSpec((2, bn, hf), lambda i: (0, i, 0)),
            pl.BlockSpec((f, f), lambda i: (0, 0)),
            pl.BlockSpec((1, f), lambda i: (0, 0)),
        ],
        out_specs=pl.BlockSpec((bn, f), lambda i: (i, 0)),
        out_shape=jax.ShapeDtypeStruct((n, f), jnp.float32),
    )(sc_out, w_ffw, b_ffw2)


def _lo_f32(v):
    return plsc.bitcast(v << jnp.uint32(16), jnp.float32)


def _hi_f32(v):
    return plsc.bitcast(v & jnp.uint32(0xFFFF0000), jnp.float32)


def _edge_sc_kernel(n, f, e, qs2, pr2, cc3, dst3, src3, init2):
    hf = f // 2
    per_tile = e // _NS
    n_chunks = per_tile // _CHUNK
    rows = n // _NS
    prc = _CHUNK // 2
    pair_per_tile = per_tile // 2

    mesh = plsc.VectorSubcoreMesh(core_axis_name="c", subcore_axis_name="s")

    @functools.partial(
        pl.kernel,
        out_type=jax.ShapeDtypeStruct((_NC, n, hf), jnp.float32),
        mesh=mesh,
        compiler_params=pltpu.CompilerParams(use_tc_tiling_on_sc=False,
                                             needs_layout_passes=False),
        scratch_types=[
            pltpu.VMEM((n_chunks, _CHUNK), jnp.int32),
            pltpu.VMEM((n_chunks, _CHUNK), jnp.int32),
            pltpu.VMEM((2, _CHUNK, hf), jnp.uint32),
            pltpu.VMEM((2, _CHUNK, hf), jnp.uint32),
            pltpu.VMEM((2, prc, f), jnp.uint32),
            pltpu.VMEM((2, _CHUNK, hf), jnp.float32),
            pltpu.VMEM_SHARED((n, hf), jnp.float32),
            pltpu.SemaphoreType.DMA,
            pltpu.SemaphoreType.DMA,
        ],
    )
    def body(qs_hbm, pr_hbm, cc_hbm, dst_hbm, src_hbm, init_hbm, out_hbm,
             dstv, srcv, qs_buf, pr_buf, cc_buf, m_buf, agg, gsem0, gsem1):
        core = lax.axis_index("c")
        sub = lax.axis_index("s")
        r0 = sub * rows
        sems = (gsem0, gsem1)

        qs_tab = qs_hbm.at[core]
        pr_tab = pr_hbm.at[core]
        cc_tab = cc_hbm.at[core]
        pbase = sub * pair_per_tile

        pltpu.sync_copy(dst_hbm.at[sub], dstv)
        pltpu.sync_copy(src_hbm.at[sub], srcv)
        pltpu.sync_copy(init_hbm.at[core, pl.ds(r0, rows)],
                        agg.at[pl.ds(r0, rows)])
        plsc.subcore_barrier()

        def issue(c, b):
            sem = sems[b]
            pltpu.async_copy(qs_tab.at[dstv.at[c]], qs_buf.at[b], sem)
            pltpu.async_copy(pr_tab.at[srcv.at[c]], pr_buf.at[b], sem)
            pltpu.async_copy(cc_tab.at[pl.ds(pbase + c * prc, prc)],
                             cc_buf.at[b], sem)

        def wait(b):
            sem = sems[b]
            pltpu.make_async_copy(
                qs_tab.at[pl.ds(0, _CHUNK)], qs_buf.at[b], sem).wait()
            pltpu.make_async_copy(
                pr_tab.at[pl.ds(0, _CHUNK)], pr_buf.at[b], sem).wait()
            pltpu.make_async_copy(
                cc_tab.at[pl.ds(0, prc)], cc_buf.at[b], sem).wait()

        def compute(b):
            @pl.loop(0, prc, unroll=2)
            def _(pe):
                e0 = 2 * pe
                for k in range(8):
                    eo = k // 4
                    kk = k % 4
                    sl = pl.ds(16 * kk, 16)
                    qv = qs_buf[b, e0 + eo, sl]
                    pv = pr_buf[b, e0 + eo, sl]
                    cv = cc_buf[b, pe, pl.ds(16 * k, 16)]
                    g_neg = (_lo_f32(qv) + _lo_f32(pv)) + _lo_f32(cv)
                    u = (_hi_f32(qv) + _hi_f32(pv)) + _hi_f32(cv)
                    denom = jnp.exp(g_neg) + 1.0
                    t = jnp.exp(-jnp.abs(u))
                    poly = t * (_C1 + t * (_C2 + t * (_C3 + t * (_C4
                                                                 + t * _C5))))
                    sp = jnp.maximum(u, 0.0) + poly
                    m_buf[b, e0 + eo, sl] = sp / denom

        def scatter(c, b):
            pltpu.sync_copy(m_buf.at[b], agg.at[dstv.at[c]], add=True)

        issue(0, 0)

        @pl.loop(0, n_chunks, step=2)
        def _(i):
            issue(i + 1, 1)
            wait(0)
            compute(0)
            scatter(i, 0)

            @pl.when(i + 2 < n_chunks)
            def _():
                issue(i + 2, 0)

            wait(1)
            compute(1)
            scatter(i + 1, 1)

        plsc.subcore_barrier()
        pltpu.sync_copy(agg.at[pl.ds(r0, rows)],
                        out_hbm.at[core, pl.ds(r0, rows)])

    return body(qs2, pr2, cc3, dst3, src3, init2)


def kernel(x, edge_index, edge_attr, W_in, b_in, W_f, b_f, W_s, b_s,
           W_ffw, b_ffw):
    n, f = x.shape
    hf = f // 2
    e = edge_index.shape[1]
    de = edge_attr.shape[1]
    per_tile = e // _NS
    n_chunks = per_tile // _CHUNK

    m1 = jnp.concatenate([-W_f[:f], W_s[:f]], axis=1)
    m2 = jnp.concatenate([-W_f[f:2 * f], W_s[f:2 * f]], axis=1)
    mc = jnp.concatenate([-W_f[2 * f:], W_s[2 * f:]], axis=1)
    bcv = jnp.concatenate([-b_f, b_s])

    mg = -W_f[2 * f:]
    mu = W_s[2 * f:]
    w2 = jnp.zeros((2 * de, 8 * hf), jnp.float32)
    for blk, half in enumerate([mg[:, :hf], mu[:, :hf],
                                mg[:, hf:], mu[:, hf:]]):
        w2 = w2.at[:de, (2 * blk) * hf:(2 * blk + 1) * hf].set(half)
        w2 = w2.at[de:, (2 * blk + 1) * hf:(2 * blk + 2) * hf].set(half)
    bc2 = jnp.concatenate([jnp.tile(-b_f[:hf], 2), jnp.tile(b_s[:hf], 2),
                           jnp.tile(-b_f[hf:], 2),
                           jnp.tile(b_s[hf:], 2)])[None, :]

    init2, qs2, pr2 = _node_tables(x, W_in, b_in[None, :], m1, m2)

    attrp = edge_attr.reshape(e // 2, 2 * de)
    cc3 = _edge_tables(attrp, w2, bc2, f)

    dst3 = edge_index[1].astype(jnp.int32).reshape(_NS, n_chunks, _CHUNK)
    src3 = edge_index[0].astype(jnp.int32).reshape(_NS, n_chunks, _CHUNK)

    sc_out = _edge_sc_kernel(n, f, e, qs2, pr2, cc3, dst3, src3, init2)

    return _final(sc_out, W_ffw, b_ffw[None, :])


# speedup vs baseline: 2.8628x; 2.8628x over previous
import functools

import jax
import jax.numpy as jnp
from jax import lax
from jax.experimental import pallas as pl
from jax.experimental.pallas import tpu as pltpu
from jax.experimental.pallas import tpu_sc as plsc

_NC = 2
_NS = 16

_CHUNK = 50

_C1 = 0.999434943
_C2 = -0.491347462
_C3 = 0.287824693
_C4 = -0.134133305
_C5 = 0.0313766221


def _pack_gu(g, u):
    bg = lax.bitcast_convert_type(g, jnp.uint32)
    bu = lax.bitcast_convert_type(u, jnp.uint32)
    half = jnp.uint32(0x7FFF)
    one = jnp.uint32(1)
    rg = (bg + half + ((bg >> 16) & one)) >> 16
    ru = (bu + half + ((bu >> 16) & one)) & jnp.uint32(0xFFFF0000)
    return ru | rg


def _node_tables(x, w_in, b_in2, m1, m2):
    n, f = x.shape
    hf = f // 2
    bn = 2000
    grid = (n // bn,)

    def body(x_ref, w_ref, b_ref, m1_ref, m2_ref, init_ref, qs_ref, pr_ref):
        h = jnp.maximum(
            jnp.dot(x_ref[...], w_ref[...], preferred_element_type=jnp.float32)
            + b_ref[...], 0.0)
        init_ref[0] = h[:, :hf]
        init_ref[1] = h[:, hf:]
        t1 = jnp.dot(h, m1_ref[...], preferred_element_type=jnp.float32)
        p1 = _pack_gu(t1[:, :f], t1[:, f:])
        qs_ref[0] = p1[:, :hf]
        qs_ref[1] = p1[:, hf:]
        t2 = jnp.dot(h, m2_ref[...], preferred_element_type=jnp.float32)
        p2 = _pack_gu(t2[:, :f], t2[:, f:])
        pr_ref[0] = p2[:, :hf]
        pr_ref[1] = p2[:, hf:]

    return pl.pallas_call(
        body,
        grid=grid,
        in_specs=[
            pl.BlockSpec((bn, f), lambda i: (i, 0)),
            pl.BlockSpec((f, f), lambda i: (0, 0)),
            pl.BlockSpec((1, f), lambda i: (0, 0)),
            pl.BlockSpec((f, 2 * f), lambda i: (0, 0)),
            pl.BlockSpec((f, 2 * f), lambda i: (0, 0)),
        ],
        out_specs=[
            pl.BlockSpec((2, bn, hf), lambda i: (0, i, 0)),
            pl.BlockSpec((2, bn, hf), lambda i: (0, i, 0)),
            pl.BlockSpec((2, bn, hf), lambda i: (0, i, 0)),
        ],
        out_shape=[
            jax.ShapeDtypeStruct((2, n, hf), jnp.float32),
            jax.ShapeDtypeStruct((2, n, hf), jnp.uint32),
            jax.ShapeDtypeStruct((2, n, hf), jnp.uint32),
        ],
    )(x, w_in, b_in2, m1, m2)


def _edge_tables(attrp, w2, bc2, f):
    ep, k = attrp.shape
    n_out = w2.shape[1]
    hf = f // 2
    bp = 4000
    grid = (ep // bp,)

    def body(a_ref, w_ref, b_ref, o_ref):
        t = jnp.dot(a_ref[...].astype(jnp.bfloat16),
                    w_ref[...].astype(jnp.bfloat16),
                    preferred_element_type=jnp.float32) + b_ref[...]
        o_ref[0] = _pack_gu(t[:, 0:f], t[:, f:2 * f])
        o_ref[1] = _pack_gu(t[:, 2 * f:3 * f], t[:, 3 * f:4 * f])

    return pl.pallas_call(
        body,
        grid=grid,
        in_specs=[
            pl.BlockSpec((bp, k), lambda i: (i, 0)),
            pl.BlockSpec((k, n_out), lambda i: (0, 0)),
            pl.BlockSpec((1, n_out), lambda i: (0, 0)),
        ],
        out_specs=pl.BlockSpec((2, bp, f), lambda i: (0, i, 0)),
        out_shape=jax.ShapeDtypeStruct((2, ep, f), jnp.uint32),
    )(attrp, w2, bc2)


def _final(sc_out, w_ffw, b_ffw2):
    _, n, hf = sc_out.shape
    f = 2 * hf
    bn = 2000
    grid = (n // bn,)

    def body(a_ref, w_ref, b_ref, o_ref):
        hh = jnp.concatenate([a_ref[0], a_ref[1]], axis=-1)
        o_ref[...] = jnp.maximum(
            jnp.dot(hh, w_ref[...], preferred_element_type=jnp.float32)
            + b_ref[...], 0.0)

    return pl.pallas_call(
        body,
        grid=grid,
        in_specs=[
            pl.BlockSpec((2, bn, hf), lambda i: (0, i, 0)),
            pl.BlockSpec((f, f), lambda i: (0, 0)),
            pl.BlockSpec((1, f), lambda i: (0, 0)),
        ],
        out_specs=pl.BlockSpec((bn, f), lambda i: (i, 0)),
        out_shape=jax.ShapeDtypeStruct((n, f), jnp.float32),
    )(sc_out, w_ffw, b_ffw2)


def _lo_f32(v):
    return plsc.bitcast(v << jnp.uint32(16), jnp.float32)


def _hi_f32(v):
    return plsc.bitcast(v & jnp.uint32(0xFFFF0000), jnp.float32)


def _edge_sc_kernel(n, f, e, qs2, pr2, cc3, dst3, src3, init2):
    hf = f // 2
    per_tile = e // _NS
    n_chunks = per_tile // _CHUNK
    rows = n // _NS
    prc = _CHUNK // 2
    pair_per_tile = per_tile // 2

    mesh = plsc.VectorSubcoreMesh(core_axis_name="c", subcore_axis_name="s")

    @functools.partial(
        pl.kernel,
        out_type=jax.ShapeDtypeStruct((_NC, n, hf), jnp.float32),
        mesh=mesh,
        compiler_params=pltpu.CompilerParams(use_tc_tiling_on_sc=False,
                                             needs_layout_passes=False),
        scratch_types=[
            pltpu.VMEM((n_chunks, _CHUNK), jnp.int32),
            pltpu.VMEM((n_chunks, _CHUNK), jnp.int32),
            pltpu.VMEM((2, _CHUNK, hf), jnp.uint32),
            pltpu.VMEM((2, _CHUNK, hf), jnp.uint32),
            pltpu.VMEM((2, prc, f), jnp.uint32),
            pltpu.VMEM((2, _CHUNK, hf), jnp.float32),
            pltpu.VMEM_SHARED((n, hf), jnp.float32),
            pltpu.SemaphoreType.DMA,
            pltpu.SemaphoreType.DMA,
        ],
    )
    def body(qs_hbm, pr_hbm, cc_hbm, dst_hbm, src_hbm, init_hbm, out_hbm,
             dstv, srcv, qs_buf, pr_buf, cc_buf, m_buf, agg, gsem0, gsem1):
        core = lax.axis_index("c")
        sub = lax.axis_index("s")
        r0 = sub * rows
        sems = (gsem0, gsem1)

        qs_tab = qs_hbm.at[core]
        pr_tab = pr_hbm.at[core]
        cc_tab = cc_hbm.at[core]
        pbase = sub * pair_per_tile

        pltpu.sync_copy(dst_hbm.at[sub], dstv)
        pltpu.sync_copy(src_hbm.at[sub], srcv)
        pltpu.sync_copy(init_hbm.at[core, pl.ds(r0, rows)],
                        agg.at[pl.ds(r0, rows)])
        plsc.subcore_barrier()

        def issue(c, b):
            sem = sems[b]
            pltpu.async_copy(qs_tab.at[dstv.at[c]], qs_buf.at[b], sem)
            pltpu.async_copy(pr_tab.at[srcv.at[c]], pr_buf.at[b], sem)
            pltpu.async_copy(cc_tab.at[pl.ds(pbase + c * prc, prc)],
                             cc_buf.at[b], sem)

        def wait(b):
            sem = sems[b]
            pltpu.make_async_copy(
                qs_tab.at[pl.ds(0, _CHUNK)], qs_buf.at[b], sem).wait()
            pltpu.make_async_copy(
                pr_tab.at[pl.ds(0, _CHUNK)], pr_buf.at[b], sem).wait()
            pltpu.make_async_copy(
                cc_tab.at[pl.ds(0, prc)], cc_buf.at[b], sem).wait()

        def compute(b):
            @pl.loop(0, prc)
            def _(pe):
                e0 = 2 * pe
                for k in range(8):
                    eo = k // 4
                    kk = k % 4
                    sl = pl.ds(16 * kk, 16)
                    qv = qs_buf[b, e0 + eo, sl]
                    pv = pr_buf[b, e0 + eo, sl]
                    cv = cc_buf[b, pe, pl.ds(16 * k, 16)]
                    g_neg = (_lo_f32(qv) + _lo_f32(pv)) + _lo_f32(cv)
                    u = (_hi_f32(qv) + _hi_f32(pv)) + _hi_f32(cv)
                    denom = jnp.exp(g_neg) + 1.0
                    t = jnp.exp(-jnp.abs(u))
                    poly = t * (_C1 + t * (_C2 + t * (_C3 + t * (_C4
                                                                 + t * _C5))))
                    sp = jnp.maximum(u, 0.0) + poly
                    m_buf[b, e0 + eo, sl] = sp / denom

        def scatter(c, b):
            pltpu.sync_copy(m_buf.at[b], agg.at[dstv.at[c]], add=True)

        issue(0, 0)

        @pl.loop(0, n_chunks, step=2)
        def _(i):
            issue(i + 1, 1)
            wait(0)
            compute(0)
            scatter(i, 0)

            @pl.when(i + 2 < n_chunks)
            def _():
                issue(i + 2, 0)

            wait(1)
            compute(1)
            scatter(i + 1, 1)

        plsc.subcore_barrier()
        pltpu.sync_copy(agg.at[pl.ds(r0, rows)],
                        out_hbm.at[core, pl.ds(r0, rows)])

    return body(qs2, pr2, cc3, dst3, src3, init2)


def kernel(x, edge_index, edge_attr, W_in, b_in, W_f, b_f, W_s, b_s,
           W_ffw, b_ffw):
    n, f = x.shape
    hf = f // 2
    e = edge_index.shape[1]
    de = edge_attr.shape[1]
    per_tile = e // _NS
    n_chunks = per_tile // _CHUNK

    m1 = jnp.concatenate([-W_f[:f], W_s[:f]], axis=1)
    m2 = jnp.concatenate([-W_f[f:2 * f], W_s[f:2 * f]], axis=1)
    mc = jnp.concatenate([-W_f[2 * f:], W_s[2 * f:]], axis=1)
    bcv = jnp.concatenate([-b_f, b_s])

    mg = -W_f[2 * f:]
    mu = W_s[2 * f:]
    w2 = jnp.zeros((2 * de, 8 * hf), jnp.float32)
    for blk, half in enumerate([mg[:, :hf], mu[:, :hf],
                                mg[:, hf:], mu[:, hf:]]):
        w2 = w2.at[:de, (2 * blk) * hf:(2 * blk + 1) * hf].set(half)
        w2 = w2.at[de:, (2 * blk + 1) * hf:(2 * blk + 2) * hf].set(half)
    bc2 = jnp.concatenate([jnp.tile(-b_f[:hf], 2), jnp.tile(b_s[:hf], 2),
                           jnp.tile(-b_f[hf:], 2),
                           jnp.tile(b_s[hf:], 2)])[None, :]

    init2, qs2, pr2 = _node_tables(x, W_in, b_in[None, :], m1, m2)

    attrp = edge_attr.reshape(e // 2, 2 * de)
    cc3 = _edge_tables(attrp, w2, bc2, f)

    dst3 = edge_index[1].astype(jnp.int32).reshape(_NS, n_chunks, _CHUNK)
    src3 = edge_index[0].astype(jnp.int32).reshape(_NS, n_chunks, _CHUNK)

    sc_out = _edge_sc_kernel(n, f, e, qs2, pr2, cc3, dst3, src3, init2)

    return _final(sc_out, W_ffw, b_ffw[None, :])
